# baseline (speedup 1.0000x reference)
import functools

import jax
import jax.numpy as jnp
from jax import lax
from jax.experimental import pallas as pl
from jax.experimental.pallas import tpu as pltpu
from jax.experimental.pallas import tpu_sc as plsc

N = 10000
E = 320000
D = 128
L = 20000
NS = 16
NC = 2
NPAD = 10240
EPT = E // NS
CH = 128
EPTP = 20480
IBLK = 10
NGRP = EPTP // (IBLK * CH)
RPT = NPAD // NS
LPT = L // (NC * NS)
LPAD = 640
LCH = LPAD // CH


def _agg_body(with_counts, *refs):
    if with_counts:
        (xcat, gidx, sidx, sums, cnt128, acc_sh,
         gbuf, sbuf, gvec0, svec0, gvec1, svec1,
         rows0, rows1, gsem0, gsem1, ssem0, ssem1) = refs
    else:
        (xcat, gidx, sidx, sums, acc_sh,
         gbuf, sbuf, gvec0, svec0, gvec1, svec1,
         rows0, rows1, gsem0, gsem1, ssem0, ssem1) = refs
        cnt128 = None
    c = lax.axis_index("c")
    s = lax.axis_index("s")

    zero16 = jnp.zeros((16,), jnp.float32)
    one16 = jnp.full((16,), 1.0, jnp.float32)
    base = s * RPT
    d_out = 1 - c

    def fill_rows(rows, val16):
        def fr(r, _):
            for q in range(D // 16):
                rows[r, pl.ds(q * 16, 16)] = val16
            return 0
        lax.fori_loop(0, CH, fr, 0)

    def zero_acc():
        def za(j, _):
            pltpu.sync_copy(rows0, acc_sh.at[pl.ds(base + j * CH, CH)])
            return 0
        lax.fori_loop(0, RPT // CH, za, 0)

    def copy_acc_out(dst):
        def co(j, _):
            r0 = base + j * CH
            pltpu.sync_copy(acc_sh.at[pl.ds(r0, CH)], rows0)
            pltpu.sync_copy(rows0, dst.at[d_out, pl.ds(r0, CH)])
            return 0
        lax.fori_loop(0, RPT // CH, co, 0)

    def stage(vg, vs, jj):
        for q in range(CH // 16):
            sl = pl.ds(q * 16, 16)
            vg[sl] = gbuf[jj, sl]
            vs[sl] = sbuf[jj, sl]

    def stage_s(vs, jj):
        for q in range(CH // 16):
            sl = pl.ds(q * 16, 16)
            vs[sl] = sbuf[jj, sl]

    if with_counts:
        fill_rows(rows0, zero16)
        zero_acc()
        plsc.subcore_barrier()
        fill_rows(rows1, one16)

        def cgrp(jg, _):
            pltpu.sync_copy(sidx.at[c, s, jg], sbuf)
            stage_s(svec0, 0)
            pltpu.async_copy(rows1, acc_sh.at[svec0], ssem0, add=True)

            def cpair(p, _):
                stage_s(svec1, 2 * p + 1)
                pltpu.async_copy(rows1, acc_sh.at[svec1], ssem1, add=True)
                pltpu.make_async_copy(rows1, acc_sh.at[svec0], ssem0).wait()

                @pl.when(p < IBLK // 2 - 1)
                def _():
                    stage_s(svec0, 2 * p + 2)
                    pltpu.async_copy(rows1, acc_sh.at[svec0], ssem0, add=True)
                pltpu.make_async_copy(rows1, acc_sh.at[svec1], ssem1).wait()
                return 0
            lax.fori_loop(0, IBLK // 2, cpair, 0)
            return 0
        lax.fori_loop(0, NGRP, cgrp, 0)
        plsc.subcore_barrier()
        copy_acc_out(cnt128)
        plsc.subcore_barrier()

    fill_rows(rows0, zero16)
    zero_acc()
    plsc.subcore_barrier()

    def grp(jg, _):
        pltpu.sync_copy(gidx.at[c, s, jg], gbuf)
        pltpu.sync_copy(sidx.at[c, s, jg], sbuf)
        stage(gvec0, svec0, 0)
        pltpu.async_copy(xcat.at[gvec0], rows0, gsem0)

        def pair(p, _):
            @pl.when(p > 0)
            def _():
                pltpu.make_async_copy(rows1, acc_sh.at[svec1], ssem1).wait()
            stage(gvec1, svec1, 2 * p + 1)
            pltpu.async_copy(xcat.at[gvec1], rows1, gsem1)
            pltpu.make_async_copy(xcat.at[gvec0], rows0, gsem0).wait()
            pltpu.async_copy(rows0, acc_sh.at[svec0], ssem0, add=True)
            pltpu.make_async_copy(xcat.at[gvec1], rows1, gsem1).wait()
            pltpu.async_copy(rows1, acc_sh.at[svec1], ssem1, add=True)

            @pl.when(p < IBLK // 2 - 1)
            def _():
                pltpu.make_async_copy(rows0, acc_sh.at[svec0], ssem0).wait()
                stage(gvec0, svec0, 2 * p + 2)
                pltpu.async_copy(xcat.at[gvec0], rows0, gsem0)
            return 0
        lax.fori_loop(0, IBLK // 2, pair, 0)
        pltpu.make_async_copy(rows0, acc_sh.at[svec0], ssem0).wait()
        pltpu.make_async_copy(rows1, acc_sh.at[svec1], ssem1).wait()
        return 0
    lax.fori_loop(0, NGRP, grp, 0)

    plsc.subcore_barrier()
    copy_acc_out(sums)


def _make_agg(with_counts):
    out_type = [jax.ShapeDtypeStruct((NC, NPAD, D), jnp.float32)]
    scratch = [
        pltpu.VMEM_SHARED((NPAD, D), jnp.float32),
        pltpu.VMEM((IBLK, CH), jnp.int32),
        pltpu.VMEM((IBLK, CH), jnp.int32),
        pltpu.VMEM((CH,), jnp.int32),
        pltpu.VMEM((CH,), jnp.int32),
        pltpu.VMEM((CH,), jnp.int32),
        pltpu.VMEM((CH,), jnp.int32),
        pltpu.VMEM((CH, D), jnp.float32),
        pltpu.VMEM((CH, D), jnp.float32),
        pltpu.SemaphoreType.DMA,
        pltpu.SemaphoreType.DMA,
        pltpu.SemaphoreType.DMA,
    ]
    if with_counts:
        out_type.append(jax.ShapeDtypeStruct((NC, NPAD, D), jnp.float32))
    scratch.append(pltpu.SemaphoreType.DMA)
    mesh = plsc.VectorSubcoreMesh(core_axis_name="c", subcore_axis_name="s")
    return pl.kernel(
        functools.partial(_agg_body, with_counts),
        out_type=tuple(out_type),
        mesh=mesh,
        scratch_types=tuple(scratch),
    )


def _sage_tc_body(relu, sums, cnt128, x, wl, wr, b, out):
    inv = 1.0 / jnp.maximum(cnt128[0][:, :1], 1.0)
    h = inv * jnp.dot(sums[0], wl[0], preferred_element_type=jnp.float32)
    h = h + b[0]
    h = h + jnp.dot(x[0], wr[0], preferred_element_type=jnp.float32)
    if relu:
        h = jnp.maximum(h, 0.0)
    out[0] = h


def _make_sage_tc(relu, rows_blk):
    grid = (NC, NPAD // rows_blk)
    return pl.pallas_call(
        functools.partial(_sage_tc_body, relu),
        grid=grid,
        in_specs=[
            pl.BlockSpec((1, rows_blk, D), lambda d, r: (d, r, 0)),
            pl.BlockSpec((1, rows_blk, D), lambda d, r: (d, r, 0)),
            pl.BlockSpec((1, rows_blk, D), lambda d, r: (d, r, 0)),
            pl.BlockSpec((1, D, D), lambda d, r: (d, 0, 0)),
            pl.BlockSpec((1, D, D), lambda d, r: (d, 0, 0)),
            pl.BlockSpec((1, 1, D), lambda d, r: (d, 0, 0)),
        ],
        out_specs=pl.BlockSpec((1, rows_blk, D), lambda d, r: (d, r, 0)),
        out_shape=jax.ShapeDtypeStruct((NC, NPAD, D), jnp.float32),
    )


def _prod_body(hcat, eli, out, aidx, bidx, avec, bvec, arows, brows, sem):
    c = lax.axis_index("c")
    s = lax.axis_index("s")
    t = c * NS + s
    pltpu.sync_copy(eli.at[0, t], aidx)
    pltpu.sync_copy(eli.at[1, t], bidx)

    def chunk(ch, _):
        for q in range(CH // 16):
            sl = pl.ds(q * 16, 16)
            avec[sl] = aidx[ch, sl]
            bvec[sl] = bidx[ch, sl]
        pltpu.async_copy(hcat.at[avec], arows, sem).wait()
        pltpu.async_copy(hcat.at[bvec], brows, sem).wait()

        def row(r, _):
            for q in range(D // 16):
                sl = pl.ds(q * 16, 16)
                arows[r, sl] = arows[r, sl] * brows[r, sl]
            return 0
        lax.fori_loop(0, CH, row, 0)
        pltpu.sync_copy(arows, out.at[pl.ds(t * LPAD + ch * CH, CH)])
        return 0
    lax.fori_loop(0, LCH, chunk, 0)


def _make_prod():
    mesh = plsc.VectorSubcoreMesh(core_axis_name="c", subcore_axis_name="s")
    return pl.kernel(
        _prod_body,
        out_type=jax.ShapeDtypeStruct((NC * NS * LPAD, D), jnp.float32),
        mesh=mesh,
        scratch_types=(
            pltpu.VMEM((LCH, CH), jnp.int32),
            pltpu.VMEM((LCH, CH), jnp.int32),
            pltpu.VMEM((CH,), jnp.int32),
            pltpu.VMEM((CH,), jnp.int32),
            pltpu.VMEM((CH, D), jnp.float32),
            pltpu.VMEM((CH, D), jnp.float32),
            pltpu.SemaphoreType.DMA,
        ),
    )


def _sig_body(prod, out):
    out[...] = jax.nn.sigmoid(jnp.sum(prod[...], axis=1, keepdims=True))


def _make_sig(rows_blk):
    return pl.pallas_call(
        _sig_body,
        grid=(NC * NS * LPAD // rows_blk,),
        in_specs=[pl.BlockSpec((rows_blk, D), lambda r: (r, 0))],
        out_specs=pl.BlockSpec((rows_blk, 1), lambda r: (r, 0)),
        out_shape=jax.ShapeDtypeStruct((NC * NS * LPAD, 1), jnp.float32),
    )


def kernel(src_node_id, tgt_node_id, edge_index, edge_label_index,
           emb_src, emb_tgt,
           W1_to_l, b1_to, W1_to_r, W1_rev_l, b1_rev, W1_rev_r,
           W2_to_l, b2_to, W2_to_r, W2_rev_l, b2_rev, W2_rev_r):
    src_e = edge_index[0]
    dst_e = edge_index[1]

    pad_e = ((0, 0), (0, EPTP - EPT))
    g0 = jnp.pad(src_e.reshape(NS, EPT), pad_e)
    g1 = jnp.pad(dst_e.reshape(NS, EPT) + NPAD, pad_e)
    s0 = jnp.pad(dst_e.reshape(NS, EPT), pad_e, constant_values=NPAD - 1)
    s1 = jnp.pad(src_e.reshape(NS, EPT), pad_e, constant_values=NPAD - 1)
    gidx = jnp.stack([g0, g1]).reshape(NC, NS, NGRP, IBLK, CH)
    sidx = jnp.stack([s0, s1]).reshape(NC, NS, NGRP, IBLK, CH)

    pad = ((0, NPAD - N), (0, 0))
    xcat1 = jnp.concatenate(
        [jnp.pad(emb_src, pad), jnp.pad(emb_tgt, pad)], axis=0)

    agg1 = _make_agg(True)
    sums1, cnt128 = agg1(xcat1, gidx, sidx)

    Wl1 = jnp.stack([W1_rev_l, W1_to_l])
    Wr1 = jnp.stack([W1_rev_r, W1_to_r])
    b1 = jnp.stack([b1_rev, b1_to])[:, None, :]
    x1 = xcat1.reshape(NC, NPAD, D)
    h1 = _make_sage_tc(True, 2048)(sums1, cnt128, x1, Wl1, Wr1, b1)

    agg2 = _make_agg(False)
    (sums2,) = agg2(h1.reshape(NC * NPAD, D), gidx, sidx)

    Wl2 = jnp.stack([W2_rev_l, W2_to_l])
    Wr2 = jnp.stack([W2_rev_r, W2_to_r])
    b2 = jnp.stack([b2_rev, b2_to])[:, None, :]
    h2 = _make_sage_tc(False, 2048)(sums2, cnt128, h1, Wl2, Wr2, b2)

    ia = jnp.pad(edge_label_index[0].reshape(NC * NS, LPT),
                 ((0, 0), (0, LPAD - LPT)))
    ib = jnp.pad((edge_label_index[1] + NPAD).reshape(NC * NS, LPT),
                 ((0, 0), (0, LPAD - LPT)))
    eli = jnp.stack([ia, ib]).reshape(2, NC * NS, LCH, CH)

    prod = _make_prod()(h2.reshape(NC * NPAD, D), eli)
    dots = _make_sig(2048)(prod)
    return dots.reshape(NC * NS, LPAD)[:, :LPT].reshape(L)


# speedup vs baseline: 1.0610x; 1.0610x over previous
import functools

import jax
import jax.numpy as jnp
from jax import lax
from jax.experimental import pallas as pl
from jax.experimental.pallas import tpu as pltpu
from jax.experimental.pallas import tpu_sc as plsc

N = 10000
E = 320000
D = 128
L = 20000
NS = 16
NC = 2
NPAD = 10240
EPT = E // NS
CH = 128
EPTP = 20480
IBLK = 10
NGRP = EPTP // (IBLK * CH)
RPT = NPAD // NS
LPT = L // (NC * NS)
LPAD = 640
LCH = LPAD // CH


def _agg_body(with_counts, *refs):
    if with_counts:
        (xcat, gidx, sidx, sums, cnt128, acc_sh,
         gbuf, sbuf, gvec0, svec0, gvec1, svec1,
         rows0, rows1, gsem0, gsem1, ssem0, ssem1) = refs
    else:
        (xcat, gidx, sidx, sums, acc_sh,
         gbuf, sbuf, gvec0, svec0, gvec1, svec1,
         rows0, rows1, gsem0, gsem1, ssem0, ssem1) = refs
        cnt128 = None
    c = lax.axis_index("c")
    s = lax.axis_index("s")

    zero16 = jnp.zeros((16,), jnp.float32)
    one16 = jnp.full((16,), 1.0, jnp.float32)
    base = s * RPT
    d_out = 1 - c

    def fill_rows(rows, val16):
        def fr(r, _):
            for q in range(D // 16):
                rows[r, pl.ds(q * 16, 16)] = val16
            return 0
        lax.fori_loop(0, CH, fr, 0)

    def zero_acc():
        def za(j, _):
            pltpu.sync_copy(rows0, acc_sh.at[pl.ds(base + j * CH, CH)])
            return 0
        lax.fori_loop(0, RPT // CH, za, 0)

    def copy_acc_out(dst):
        def co(j, _):
            r0 = base + j * CH
            pltpu.sync_copy(acc_sh.at[pl.ds(r0, CH)], rows0)
            pltpu.sync_copy(rows0, dst.at[d_out, pl.ds(r0, CH)])
            return 0
        lax.fori_loop(0, RPT // CH, co, 0)

    def stage(vg, vs, jj):
        for q in range(CH // 16):
            sl = pl.ds(q * 16, 16)
            vg[sl] = gbuf[jj, sl]
            vs[sl] = sbuf[jj, sl]

    def stage_s(vs, jj):
        for q in range(CH // 16):
            sl = pl.ds(q * 16, 16)
            vs[sl] = sbuf[jj, sl]

    if with_counts:
        fill_rows(rows0, zero16)
        zero_acc()
        plsc.subcore_barrier()
        fill_rows(rows1, one16)

        def cgrp(jg, _):
            pltpu.sync_copy(sidx.at[c, s, jg], sbuf)
            stage_s(svec0, 0)
            pltpu.async_copy(rows1, acc_sh.at[svec0], ssem0, add=True)

            def cpair(p, _):
                stage_s(svec1, 2 * p + 1)
                pltpu.async_copy(rows1, acc_sh.at[svec1], ssem1, add=True)
                pltpu.make_async_copy(rows1, acc_sh.at[svec0], ssem0).wait()

                @pl.when(p < IBLK // 2 - 1)
                def _():
                    stage_s(svec0, 2 * p + 2)
                    pltpu.async_copy(rows1, acc_sh.at[svec0], ssem0, add=True)
                pltpu.make_async_copy(rows1, acc_sh.at[svec1], ssem1).wait()
                return 0
            lax.fori_loop(0, IBLK // 2, cpair, 0)
            return 0
        lax.fori_loop(0, NGRP, cgrp, 0)
        plsc.subcore_barrier()
        copy_acc_out(cnt128)
        plsc.subcore_barrier()

    fill_rows(rows0, zero16)
    zero_acc()
    plsc.subcore_barrier()

    def grp(jg, _):
        pltpu.sync_copy(gidx.at[c, s, jg], gbuf)
        pltpu.sync_copy(sidx.at[c, s, jg], sbuf)
        stage(gvec0, svec0, 0)
        pltpu.async_copy(xcat.at[gvec0], rows0, gsem0)

        def pair(p, _):
            stage(gvec1, svec1, 2 * p + 1)
            pltpu.async_copy(xcat.at[gvec1], rows1, gsem1)
            pltpu.make_async_copy(xcat.at[gvec0], rows0, gsem0).wait()
            pltpu.sync_copy(rows0, acc_sh.at[svec0], add=True)

            @pl.when(p < IBLK // 2 - 1)
            def _():
                stage(gvec0, svec0, 2 * p + 2)
                pltpu.async_copy(xcat.at[gvec0], rows0, gsem0)
            pltpu.make_async_copy(xcat.at[gvec1], rows1, gsem1).wait()
            pltpu.sync_copy(rows1, acc_sh.at[svec1], add=True)
            return 0
        lax.fori_loop(0, IBLK // 2, pair, 0)
        return 0
    lax.fori_loop(0, NGRP, grp, 0)

    plsc.subcore_barrier()
    copy_acc_out(sums)


def _make_agg(with_counts):
    out_type = [jax.ShapeDtypeStruct((NC, NPAD, D), jnp.float32)]
    scratch = [
        pltpu.VMEM_SHARED((NPAD, D), jnp.float32),
        pltpu.VMEM((IBLK, CH), jnp.int32),
        pltpu.VMEM((IBLK, CH), jnp.int32),
        pltpu.VMEM((CH,), jnp.int32),
        pltpu.VMEM((CH,), jnp.int32),
        pltpu.VMEM((CH,), jnp.int32),
        pltpu.VMEM((CH,), jnp.int32),
        pltpu.VMEM((CH, D), jnp.float32),
        pltpu.VMEM((CH, D), jnp.float32),
        pltpu.SemaphoreType.DMA,
        pltpu.SemaphoreType.DMA,
        pltpu.SemaphoreType.DMA,
    ]
    if with_counts:
        out_type.append(jax.ShapeDtypeStruct((NC, NPAD, D), jnp.float32))
    scratch.append(pltpu.SemaphoreType.DMA)
    mesh = plsc.VectorSubcoreMesh(core_axis_name="c", subcore_axis_name="s")
    return pl.kernel(
        functools.partial(_agg_body, with_counts),
        out_type=tuple(out_type),
        mesh=mesh,
        scratch_types=tuple(scratch),
    )


def _sage_tc_body(relu, sums, cnt128, x, wl, wr, b, out):
    inv = 1.0 / jnp.maximum(cnt128[0][:, :1], 1.0)
    h = inv * jnp.dot(sums[0], wl[0], preferred_element_type=jnp.float32)
    h = h + b[0]
    h = h + jnp.dot(x[0], wr[0], preferred_element_type=jnp.float32)
    if relu:
        h = jnp.maximum(h, 0.0)
    out[0] = h


def _make_sage_tc(relu, rows_blk):
    grid = (NC, NPAD // rows_blk)
    return pl.pallas_call(
        functools.partial(_sage_tc_body, relu),
        grid=grid,
        in_specs=[
            pl.BlockSpec((1, rows_blk, D), lambda d, r: (d, r, 0)),
            pl.BlockSpec((1, rows_blk, D), lambda d, r: (d, r, 0)),
            pl.BlockSpec((1, rows_blk, D), lambda d, r: (d, r, 0)),
            pl.BlockSpec((1, D, D), lambda d, r: (d, 0, 0)),
            pl.BlockSpec((1, D, D), lambda d, r: (d, 0, 0)),
            pl.BlockSpec((1, 1, D), lambda d, r: (d, 0, 0)),
        ],
        out_specs=pl.BlockSpec((1, rows_blk, D), lambda d, r: (d, r, 0)),
        out_shape=jax.ShapeDtypeStruct((NC, NPAD, D), jnp.float32),
    )


def _prod_body(hcat, eli, out, aidx, bidx, avec, bvec, arows, brows, sem):
    c = lax.axis_index("c")
    s = lax.axis_index("s")
    t = c * NS + s
    pltpu.sync_copy(eli.at[0, t], aidx)
    pltpu.sync_copy(eli.at[1, t], bidx)

    def chunk(ch, _):
        for q in range(CH // 16):
            sl = pl.ds(q * 16, 16)
            avec[sl] = aidx[ch, sl]
            bvec[sl] = bidx[ch, sl]
        pltpu.async_copy(hcat.at[avec], arows, sem).wait()
        pltpu.async_copy(hcat.at[bvec], brows, sem).wait()

        def row(r, _):
            for q in range(D // 16):
                sl = pl.ds(q * 16, 16)
                arows[r, sl] = arows[r, sl] * brows[r, sl]
            return 0
        lax.fori_loop(0, CH, row, 0)
        pltpu.sync_copy(arows, out.at[pl.ds(t * LPAD + ch * CH, CH)])
        return 0
    lax.fori_loop(0, LCH, chunk, 0)


def _make_prod():
    mesh = plsc.VectorSubcoreMesh(core_axis_name="c", subcore_axis_name="s")
    return pl.kernel(
        _prod_body,
        out_type=jax.ShapeDtypeStruct((NC * NS * LPAD, D), jnp.float32),
        mesh=mesh,
        scratch_types=(
            pltpu.VMEM((LCH, CH), jnp.int32),
            pltpu.VMEM((LCH, CH), jnp.int32),
            pltpu.VMEM((CH,), jnp.int32),
            pltpu.VMEM((CH,), jnp.int32),
            pltpu.VMEM((CH, D), jnp.float32),
            pltpu.VMEM((CH, D), jnp.float32),
            pltpu.SemaphoreType.DMA,
        ),
    )


def _sig_body(prod, out):
    out[...] = jax.nn.sigmoid(jnp.sum(prod[...], axis=1, keepdims=True))


def _make_sig(rows_blk):
    return pl.pallas_call(
        _sig_body,
        grid=(NC * NS * LPAD // rows_blk,),
        in_specs=[pl.BlockSpec((rows_blk, D), lambda r: (r, 0))],
        out_specs=pl.BlockSpec((rows_blk, 1), lambda r: (r, 0)),
        out_shape=jax.ShapeDtypeStruct((NC * NS * LPAD, 1), jnp.float32),
    )


def kernel(src_node_id, tgt_node_id, edge_index, edge_label_index,
           emb_src, emb_tgt,
           W1_to_l, b1_to, W1_to_r, W1_rev_l, b1_rev, W1_rev_r,
           W2_to_l, b2_to, W2_to_r, W2_rev_l, b2_rev, W2_rev_r):
    src_e = edge_index[0]
    dst_e = edge_index[1]

    pad_e = ((0, 0), (0, EPTP - EPT))
    g0 = jnp.pad(src_e.reshape(NS, EPT), pad_e)
    g1 = jnp.pad(dst_e.reshape(NS, EPT) + NPAD, pad_e)
    s0 = jnp.pad(dst_e.reshape(NS, EPT), pad_e, constant_values=NPAD - 1)
    s1 = jnp.pad(src_e.reshape(NS, EPT), pad_e, constant_values=NPAD - 1)
    gidx = jnp.stack([g0, g1]).reshape(NC, NS, NGRP, IBLK, CH)
    sidx = jnp.stack([s0, s1]).reshape(NC, NS, NGRP, IBLK, CH)

    pad = ((0, NPAD - N), (0, 0))
    xcat1 = jnp.concatenate(
        [jnp.pad(emb_src, pad), jnp.pad(emb_tgt, pad)], axis=0)

    agg1 = _make_agg(True)
    sums1, cnt128 = agg1(xcat1, gidx, sidx)

    Wl1 = jnp.stack([W1_rev_l, W1_to_l])
    Wr1 = jnp.stack([W1_rev_r, W1_to_r])
    b1 = jnp.stack([b1_rev, b1_to])[:, None, :]
    x1 = xcat1.reshape(NC, NPAD, D)
    h1 = _make_sage_tc(True, 2048)(sums1, cnt128, x1, Wl1, Wr1, b1)

    agg2 = _make_agg(False)
    (sums2,) = agg2(h1.reshape(NC * NPAD, D), gidx, sidx)

    Wl2 = jnp.stack([W2_rev_l, W2_to_l])
    Wr2 = jnp.stack([W2_rev_r, W2_to_r])
    b2 = jnp.stack([b2_rev, b2_to])[:, None, :]
    h2 = _make_sage_tc(False, 2048)(sums2, cnt128, h1, Wl2, Wr2, b2)

    ia = jnp.pad(edge_label_index[0].reshape(NC * NS, LPT),
                 ((0, 0), (0, LPAD - LPT)))
    ib = jnp.pad((edge_label_index[1] + NPAD).reshape(NC * NS, LPT),
                 ((0, 0), (0, LPAD - LPT)))
    eli = jnp.stack([ia, ib]).reshape(2, NC * NS, LCH, CH)

    prod = _make_prod()(h2.reshape(NC * NPAD, D), eli)
    dots = _make_sig(2048)(prod)
    return dots.reshape(NC * NS, LPAD)[:, :LPT].reshape(L)


# speedup vs baseline: 2.4182x; 2.2791x over previous
import functools

import jax
import jax.numpy as jnp
from jax import lax
from jax.experimental import pallas as pl
from jax.experimental.pallas import tpu as pltpu
from jax.experimental.pallas import tpu_sc as plsc

N = 10000
E = 320000
D = 128
L = 20000
NS = 16
NC = 2
NPAD = 10240
EPT = E // NS
CH = 80
EPTP = EPT
IBLK = 10
NGRP = EPTP // (IBLK * CH)
RPT = NPAD // NS
LPT = L // (NC * NS)
LPAD = 640
LCH = LPAD // CH


def _agg_body(with_counts, *refs):
    if with_counts:
        (xcat, gidx, sidx, sums, cnt128, acc_sh,
         gbuf, sbuf, gvec0, svec0, gvec1, svec1,
         rows0, rows1, gsem0, gsem1, ssem0, ssem1) = refs
    else:
        (xcat, gidx, sidx, sums, acc_sh,
         gbuf, sbuf, gvec0, svec0, gvec1, svec1,
         rows0, rows1, gsem0, gsem1, ssem0, ssem1) = refs
        cnt128 = None
    c = lax.axis_index("c")
    s = lax.axis_index("s")

    zero16 = jnp.zeros((16,), jnp.float32)
    one16 = jnp.full((16,), 1.0, jnp.float32)
    base = s * RPT
    d_out = 1 - c

    def fill_rows(rows, val16):
        def fr(r, _):
            for q in range(D // 16):
                rows[r, pl.ds(q * 16, 16)] = val16
            return 0
        lax.fori_loop(0, CH, fr, 0)

    def zero_acc():
        def za(j, _):
            pltpu.sync_copy(rows0, acc_sh.at[pl.ds(base + j * CH, CH)])
            return 0
        lax.fori_loop(0, RPT // CH, za, 0)

    def copy_acc_out(dst):
        def co(j, _):
            r0 = base + j * CH
            pltpu.sync_copy(acc_sh.at[pl.ds(r0, CH)], rows0)
            pltpu.sync_copy(rows0, dst.at[d_out, pl.ds(r0, CH)])
            return 0
        lax.fori_loop(0, RPT // CH, co, 0)

    def stage(vg, vs, jj):
        for q in range(CH // 16):
            sl = pl.ds(q * 16, 16)
            vg[sl] = gbuf[jj, sl]
            vs[sl] = sbuf[jj, sl]

    def stage_s(vs, jj):
        for q in range(CH // 16):
            sl = pl.ds(q * 16, 16)
            vs[sl] = sbuf[jj, sl]

    if with_counts:
        fill_rows(rows0, zero16)
        zero_acc()
        plsc.subcore_barrier()
        fill_rows(rows1, one16)

        def cgrp(jg, _):
            pltpu.sync_copy(sidx.at[c, s, jg], sbuf)
            stage_s(svec0, 0)
            pltpu.async_copy(rows1, acc_sh.at[svec0], ssem0, add=True)

            def cpair(p, _):
                stage_s(svec1, 2 * p + 1)
                pltpu.async_copy(rows1, acc_sh.at[svec1], ssem1, add=True)
                pltpu.make_async_copy(rows1, acc_sh.at[svec0], ssem0).wait()

                @pl.when(p < IBLK // 2 - 1)
                def _():
                    stage_s(svec0, 2 * p + 2)
                    pltpu.async_copy(rows1, acc_sh.at[svec0], ssem0, add=True)
                pltpu.make_async_copy(rows1, acc_sh.at[svec1], ssem1).wait()
                return 0
            lax.fori_loop(0, IBLK // 2, cpair, 0)
            return 0
        lax.fori_loop(0, NGRP, cgrp, 0)
        plsc.subcore_barrier()
        copy_acc_out(cnt128)
        plsc.subcore_barrier()

    fill_rows(rows0, zero16)
    zero_acc()
    plsc.subcore_barrier()

    def grp(jg, _):
        pltpu.sync_copy(gidx.at[c, s, jg], gbuf)
        pltpu.sync_copy(sidx.at[c, s, jg], sbuf)
        stage(gvec0, svec0, 0)
        pltpu.async_copy(xcat.at[gvec0], rows0, gsem0)

        def pair(p, _):
            stage(gvec1, svec1, 2 * p + 1)
            pltpu.async_copy(xcat.at[gvec1], rows1, gsem1)
            pltpu.make_async_copy(xcat.at[gvec0], rows0, gsem0).wait()
            pltpu.sync_copy(rows0, acc_sh.at[svec0], add=True)

            @pl.when(p < IBLK // 2 - 1)
            def _():
                stage(gvec0, svec0, 2 * p + 2)
                pltpu.async_copy(xcat.at[gvec0], rows0, gsem0)
            pltpu.make_async_copy(xcat.at[gvec1], rows1, gsem1).wait()
            pltpu.sync_copy(rows1, acc_sh.at[svec1], add=True)
            return 0
        lax.fori_loop(0, IBLK // 2, pair, 0)
        return 0
    lax.fori_loop(0, NGRP, grp, 0)

    plsc.subcore_barrier()
    copy_acc_out(sums)


def _make_agg(with_counts):
    out_type = [jax.ShapeDtypeStruct((NC, NPAD, D), jnp.float32)]
    scratch = [
        pltpu.VMEM_SHARED((NPAD, D), jnp.float32),
        pltpu.VMEM((IBLK, CH), jnp.int32),
        pltpu.VMEM((IBLK, CH), jnp.int32),
        pltpu.VMEM((CH,), jnp.int32),
        pltpu.VMEM((CH,), jnp.int32),
        pltpu.VMEM((CH,), jnp.int32),
        pltpu.VMEM((CH,), jnp.int32),
        pltpu.VMEM((CH, D), jnp.float32),
        pltpu.VMEM((CH, D), jnp.float32),
        pltpu.SemaphoreType.DMA,
        pltpu.SemaphoreType.DMA,
        pltpu.SemaphoreType.DMA,
    ]
    if with_counts:
        out_type.append(jax.ShapeDtypeStruct((NC, NPAD, D), jnp.float32))
    scratch.append(pltpu.SemaphoreType.DMA)
    mesh = plsc.VectorSubcoreMesh(core_axis_name="c", subcore_axis_name="s")
    return pl.kernel(
        functools.partial(_agg_body, with_counts),
        out_type=tuple(out_type),
        mesh=mesh,
        scratch_types=tuple(scratch),
    )


def _sage_tc_body(relu, sums, cnt128, x, wl, wr, b, out):
    inv = 1.0 / jnp.maximum(cnt128[0][:, :1], 1.0)
    h = inv * jnp.dot(sums[0], wl[0], preferred_element_type=jnp.float32)
    h = h + b[0]
    h = h + jnp.dot(x[0], wr[0], preferred_element_type=jnp.float32)
    if relu:
        h = jnp.maximum(h, 0.0)
    out[0] = h


def _make_sage_tc(relu, rows_blk):
    grid = (NC, NPAD // rows_blk)
    return pl.pallas_call(
        functools.partial(_sage_tc_body, relu),
        grid=grid,
        in_specs=[
            pl.BlockSpec((1, rows_blk, D), lambda d, r: (d, r, 0)),
            pl.BlockSpec((1, rows_blk, D), lambda d, r: (d, r, 0)),
            pl.BlockSpec((1, rows_blk, D), lambda d, r: (d, r, 0)),
            pl.BlockSpec((1, D, D), lambda d, r: (d, 0, 0)),
            pl.BlockSpec((1, D, D), lambda d, r: (d, 0, 0)),
            pl.BlockSpec((1, 1, D), lambda d, r: (d, 0, 0)),
        ],
        out_specs=pl.BlockSpec((1, rows_blk, D), lambda d, r: (d, r, 0)),
        out_shape=jax.ShapeDtypeStruct((NC, NPAD, D), jnp.float32),
    )


def _prod_body(hcat, eli, out, aidx, bidx, avec, bvec, arows, brows, sem):
    c = lax.axis_index("c")
    s = lax.axis_index("s")
    t = c * NS + s
    pltpu.sync_copy(eli.at[0, t], aidx)
    pltpu.sync_copy(eli.at[1, t], bidx)

    def chunk(ch, _):
        for q in range(CH // 16):
            sl = pl.ds(q * 16, 16)
            avec[sl] = aidx[ch, sl]
            bvec[sl] = bidx[ch, sl]
        pltpu.async_copy(hcat.at[avec], arows, sem).wait()
        pltpu.async_copy(hcat.at[bvec], brows, sem).wait()

        def row(r, _):
            for q in range(D // 16):
                sl = pl.ds(q * 16, 16)
                arows[r, sl] = arows[r, sl] * brows[r, sl]
            return 0
        lax.fori_loop(0, CH, row, 0)
        pltpu.sync_copy(arows, out.at[pl.ds(t * LPAD + ch * CH, CH)])
        return 0
    lax.fori_loop(0, LCH, chunk, 0)


def _make_prod():
    mesh = plsc.VectorSubcoreMesh(core_axis_name="c", subcore_axis_name="s")
    return pl.kernel(
        _prod_body,
        out_type=jax.ShapeDtypeStruct((NC * NS * LPAD, D), jnp.float32),
        mesh=mesh,
        scratch_types=(
            pltpu.VMEM((LCH, CH), jnp.int32),
            pltpu.VMEM((LCH, CH), jnp.int32),
            pltpu.VMEM((CH,), jnp.int32),
            pltpu.VMEM((CH,), jnp.int32),
            pltpu.VMEM((CH, D), jnp.float32),
            pltpu.VMEM((CH, D), jnp.float32),
            pltpu.SemaphoreType.DMA,
        ),
    )


def _sig_body(prod, out):
    out[...] = jax.nn.sigmoid(jnp.sum(prod[...], axis=1, keepdims=True))


def _make_sig(rows_blk):
    return pl.pallas_call(
        _sig_body,
        grid=(NC * NS * LPAD // rows_blk,),
        in_specs=[pl.BlockSpec((rows_blk, D), lambda r: (r, 0))],
        out_specs=pl.BlockSpec((rows_blk, 1), lambda r: (r, 0)),
        out_shape=jax.ShapeDtypeStruct((NC * NS * LPAD, 1), jnp.float32),
    )


def kernel(src_node_id, tgt_node_id, edge_index, edge_label_index,
           emb_src, emb_tgt,
           W1_to_l, b1_to, W1_to_r, W1_rev_l, b1_rev, W1_rev_r,
           W2_to_l, b2_to, W2_to_r, W2_rev_l, b2_rev, W2_rev_r):
    src_e = edge_index[0]
    dst_e = edge_index[1]

    pad_e = ((0, 0), (0, EPTP - EPT))
    g0 = jnp.pad(src_e.reshape(NS, EPT), pad_e)
    g1 = jnp.pad(dst_e.reshape(NS, EPT) + NPAD, pad_e)
    s0 = jnp.pad(dst_e.reshape(NS, EPT), pad_e, constant_values=NPAD - 1)
    s1 = jnp.pad(src_e.reshape(NS, EPT), pad_e, constant_values=NPAD - 1)
    gidx = jnp.stack([g0, g1]).reshape(NC, NS, NGRP, IBLK, CH)
    sidx = jnp.stack([s0, s1]).reshape(NC, NS, NGRP, IBLK, CH)

    pad = ((0, NPAD - N), (0, 0))
    xcat1 = jnp.concatenate(
        [jnp.pad(emb_src, pad), jnp.pad(emb_tgt, pad)], axis=0)

    agg1 = _make_agg(True)
    sums1, cnt128 = agg1(xcat1, gidx, sidx)

    Wl1 = jnp.stack([W1_rev_l, W1_to_l])
    Wr1 = jnp.stack([W1_rev_r, W1_to_r])
    b1 = jnp.stack([b1_rev, b1_to])[:, None, :]
    x1 = xcat1.reshape(NC, NPAD, D)
    h1 = _make_sage_tc(True, 2048)(sums1, cnt128, x1, Wl1, Wr1, b1)

    agg2 = _make_agg(False)
    (sums2,) = agg2(h1.reshape(NC * NPAD, D), gidx, sidx)

    Wl2 = jnp.stack([W2_rev_l, W2_to_l])
    Wr2 = jnp.stack([W2_rev_r, W2_to_r])
    b2 = jnp.stack([b2_rev, b2_to])[:, None, :]
    h2 = _make_sage_tc(False, 2048)(sums2, cnt128, h1, Wl2, Wr2, b2)

    ia = jnp.pad(edge_label_index[0].reshape(NC * NS, LPT),
                 ((0, 0), (0, LPAD - LPT)))
    ib = jnp.pad((edge_label_index[1] + NPAD).reshape(NC * NS, LPT),
                 ((0, 0), (0, LPAD - LPT)))
    eli = jnp.stack([ia, ib]).reshape(2, NC * NS, LCH, CH)

    prod = _make_prod()(h2.reshape(NC * NPAD, D), eli)
    dots = _make_sig(2048)(prod)
    return dots.reshape(NC * NS, LPAD)[:, :LPT].reshape(L)
